# baseline (speedup 1.0000x reference)
import numpy as np
import jax
import jax.numpy as jnp
from jax import lax
from jax.experimental import pallas as pl

B = 32
L = 128
F = 32
D_OB = 4
D_MODEL = F * D_OB
D_PE = 16
D = D_MODEL + D_PE
H = 12
HD = D // H
D_FFN = 512
N_LAYERS = 2
C = L * D_OB
BPS = 8

_TIMESCALES = np.asarray(float(L) ** np.linspace(0.0, 1.0, D_PE // 2),
                         dtype=np.float32)

_NT = (((1,), (1,)), ((), ()))


def _prop_pe_body(xt_ref, rp_ref, w1v_ref, b1v_ref, w1s_ref, b1s_ref,
                  w2v_ref, b2v_ref, w2s_ref, b2s_ref, tr_ref, tsr_ref,
                  x0_ref):
    e = (lax.broadcasted_iota(jnp.int32, (L, C), 1) // D_OB
         == lax.broadcasted_iota(jnp.int32, (L, C), 0)).astype(jnp.float32)
    xg = jnp.dot(xt_ref[...], e, preferred_element_type=jnp.float32)
    rb = jnp.broadcast_to(rp_ref[...][None], (B, F, C)).reshape(B * F, C)
    s = jax.nn.relu(xg * rb)
    y = (jax.nn.relu(lax.dot_general(s, w1v_ref[...], _NT,
                                     preferred_element_type=jnp.float32)
                     + b1v_ref[...])
         + lax.dot_general(s, w1s_ref[...], _NT,
                           preferred_element_type=jnp.float32)
         + b1s_ref[...])
    z = (jax.nn.relu(lax.dot_general(y, w2v_ref[...], _NT,
                                     preferred_element_type=jnp.float32)
                     + b2v_ref[...])
         + lax.dot_general(y, w2s_ref[...], _NT,
                           preferred_element_type=jnp.float32)
         + b2s_ref[...])
    scaled = tr_ref[...] / tsr_ref[...]
    pes = jnp.sin(scaled)
    pec = jnp.cos(scaled)
    cidx = lax.broadcasted_iota(jnp.int32, (L, C), 1)
    ridx4 = lax.broadcasted_iota(jnp.int32, (L, C), 0) * D_OB
    planes = [lax.dot_general((cidx == ridx4 + o).astype(jnp.float32), z, _NT,
                              preferred_element_type=jnp.float32)
              for o in range(D_OB)]
    rr = lax.broadcasted_iota(jnp.int32, (D_MODEL, D_MODEL), 0)
    cc = lax.broadcasted_iota(jnp.int32, (D_MODEL, D_MODEL), 1)
    ecat = (cc == (rr % F) * D_OB + rr // F).astype(jnp.float32)
    npe = D_PE // 2
    for b in range(B):
        zb = jnp.concatenate([p[:, b * F:(b + 1) * F] for p in planes], axis=1)
        ub = jnp.dot(zb, ecat, preferred_element_type=jnp.float32)
        x0_ref[b] = jnp.concatenate(
            [ub, pes[:, b * npe:(b + 1) * npe], pec[:, b * npe:(b + 1) * npe]],
            axis=1)


def _ln(t, w, b):
    mu = jnp.mean(t, axis=-1, keepdims=True)
    var = jnp.mean((t - mu) ** 2, axis=-1, keepdims=True)
    return (t - mu) / jnp.sqrt(var + 1e-5) * w + b


def _tf_body(x_ref, neg_ref, wq_ref, wk_ref, wv_ref,
             bq_ref, bk_ref, bv_ref,
             wo_ref, bo_ref, w1_ref, b1_ref, w2_ref, b2_ref,
             n1w_ref, n1b_ref, n2w_ref, n2b_ref, out_ref):
    x = x_ref[...].reshape(BPS * L, D)
    scale = 1.0 / float(np.sqrt(HD))
    col = lax.broadcasted_iota(jnp.int32, (1, D), 1)
    masks = [(col // HD == h).astype(jnp.float32) for h in range(H)]
    for l in range(N_LAYERS):
        q = lax.dot_general(x, wq_ref[l], _NT,
                            preferred_element_type=jnp.float32) + bq_ref[l]
        k = lax.dot_general(x, wk_ref[l], _NT,
                            preferred_element_type=jnp.float32) + bk_ref[l]
        v = lax.dot_general(x, wv_ref[l], _NT,
                            preferred_element_type=jnp.float32) + bv_ref[l]
        os = []
        for j in range(BPS):
            qj = q[j * L:(j + 1) * L]
            kj = k[j * L:(j + 1) * L]
            vj = v[j * L:(j + 1) * L]
            neg = neg_ref[j]
            kms = jnp.concatenate([kj * mh for mh in masks], axis=0)
            vms = jnp.concatenate([vj * mh for mh in masks], axis=0)
            s = lax.dot_general(qj, kms, _NT,
                                preferred_element_type=jnp.float32)
            s = s * scale
            ps = []
            for h in range(H):
                sh = s[:, h * L:(h + 1) * L] + neg
                m = jnp.max(sh, axis=-1, keepdims=True)
                e = jnp.exp(sh - m)
                ps.append(e / jnp.sum(e, axis=-1, keepdims=True))
            p = jnp.concatenate(ps, axis=1)
            os.append(jnp.dot(p, vms, preferred_element_type=jnp.float32))
        o = jnp.concatenate(os, axis=0)
        a = lax.dot_general(o, wo_ref[l], _NT,
                            preferred_element_type=jnp.float32) + bo_ref[l]
        x = _ln(x + a, n1w_ref[l], n1b_ref[l])
        f = lax.dot_general(
            jax.nn.relu(
                lax.dot_general(x, w1_ref[l], _NT,
                                preferred_element_type=jnp.float32)
                + b1_ref[l]),
            w2_ref[l], _NT, preferred_element_type=jnp.float32) + b2_ref[l]
        x = _ln(x + f, n2w_ref[l], n2b_ref[l])
    for j in range(BPS):
        out_ref[:, j, :] = x[j * L:(j + 1) * L]


def kernel(X, timestamps, lengths, R_u, op1_vw, op1_vb, op1_sw, op1_sb,
           op2_vw, op2_vb, op2_sw, op2_sb, in_proj_w, in_proj_b,
           out_proj_w, out_proj_b, lin1_w, lin1_b, lin2_w, lin2_b,
           norm1_w, norm1_b, norm2_w, norm2_b):
    f32 = jnp.float32

    xt = X.transpose(0, 2, 1).reshape(B * F, L)
    rp = jnp.broadcast_to(R_u.reshape(F, D_OB)[:, None, :],
                          (F, L, D_OB)).reshape(F, C)
    times_rep = jnp.repeat(timestamps.transpose(1, 0), D_PE // 2, axis=1)
    ts_rep = jnp.tile(jnp.asarray(_TIMESCALES).reshape(1, D_PE // 2), (1, B))

    x0 = pl.pallas_call(
        _prop_pe_body,
        out_shape=jax.ShapeDtypeStruct((B, L, D), f32),
    )(xt, rp,
      op1_vw, op1_vb.reshape(1, C), op1_sw, op1_sb.reshape(1, C),
      op2_vw, op2_vb.reshape(1, C), op2_sw, op2_sb.reshape(1, C),
      times_rep, ts_rep)

    mask = jnp.arange(L)[None, :] >= lengths
    neg = jnp.where(mask, jnp.float32(-1e30), jnp.float32(0.0))
    neg3 = neg.reshape(B, 1, L)

    wq = in_proj_w[:, 0 * D:1 * D, :]
    wk = in_proj_w[:, 1 * D:2 * D, :]
    wv = in_proj_w[:, 2 * D:3 * D, :]
    bq = in_proj_b[:, 0 * D:1 * D].reshape(N_LAYERS, 1, D)
    bk = in_proj_b[:, 1 * D:2 * D].reshape(N_LAYERS, 1, D)
    bv = in_proj_b[:, 2 * D:3 * D].reshape(N_LAYERS, 1, D)

    full = lambda shape: pl.BlockSpec(shape, lambda b: (0,) * len(shape))
    xout = pl.pallas_call(
        _tf_body,
        grid=(B // BPS,),
        in_specs=[
            pl.BlockSpec((BPS, L, D), lambda b: (b, 0, 0)),
            pl.BlockSpec((BPS, 1, L), lambda b: (b, 0, 0)),
            full((N_LAYERS, D, D)), full((N_LAYERS, D, D)), full((N_LAYERS, D, D)),
            full((N_LAYERS, 1, D)), full((N_LAYERS, 1, D)), full((N_LAYERS, 1, D)),
            full((N_LAYERS, D, D)), full((N_LAYERS, 1, D)),
            full((N_LAYERS, D_FFN, D)), full((N_LAYERS, 1, D_FFN)),
            full((N_LAYERS, D, D_FFN)), full((N_LAYERS, 1, D)),
            full((N_LAYERS, 1, D)), full((N_LAYERS, 1, D)),
            full((N_LAYERS, 1, D)), full((N_LAYERS, 1, D)),
        ],
        out_specs=pl.BlockSpec((L, BPS, D), lambda b: (0, b, 0)),
        out_shape=jax.ShapeDtypeStruct((L, B, D), f32),
    )(x0, neg3, wq, wk, wv, bq, bk, bv,
      out_proj_w, out_proj_b.reshape(N_LAYERS, 1, D),
      lin1_w, lin1_b.reshape(N_LAYERS, 1, D_FFN),
      lin2_w, lin2_b.reshape(N_LAYERS, 1, D),
      norm1_w.reshape(N_LAYERS, 1, D), norm1_b.reshape(N_LAYERS, 1, D),
      norm2_w.reshape(N_LAYERS, 1, D), norm2_b.reshape(N_LAYERS, 1, D))

    return xout, mask


# speedup vs baseline: 1.0131x; 1.0131x over previous
import numpy as np
import jax
import jax.numpy as jnp
from jax import lax
from jax.experimental import pallas as pl

B = 32
L = 128
F = 32
D_OB = 4
D_MODEL = F * D_OB
D_PE = 16
D = D_MODEL + D_PE
H = 12
HD = D // H
D_FFN = 512
N_LAYERS = 2
C = L * D_OB
BPS = 8
NR = BPS * F
NPE = D_PE // 2

_TIMESCALES = np.asarray(float(L) ** np.linspace(0.0, 1.0, NPE),
                         dtype=np.float32)

_NT = (((1,), (1,)), ((), ()))


def _ln(t, w, b):
    mu = jnp.mean(t, axis=-1, keepdims=True)
    var = jnp.mean((t - mu) ** 2, axis=-1, keepdims=True)
    return (t - mu) / jnp.sqrt(var + 1e-5) * w + b


def _body(xt_ref, rp_ref, w1v_ref, b1v_ref, w1s_ref, b1s_ref,
          w2v_ref, b2v_ref, w2s_ref, b2s_ref, tr_ref, tsr_ref, neg_ref,
          wq_ref, wk_ref, wv_ref, bq_ref, bk_ref, bv_ref,
          wo_ref, bo_ref, w1_ref, b1_ref, w2_ref, b2_ref,
          n1w_ref, n1b_ref, n2w_ref, n2b_ref, out_ref):
    f32 = jnp.float32
    e = (lax.broadcasted_iota(jnp.int32, (L, C), 1) // D_OB
         == lax.broadcasted_iota(jnp.int32, (L, C), 0)).astype(f32)
    xg = jnp.dot(xt_ref[...], e, preferred_element_type=f32)
    rb = jnp.broadcast_to(rp_ref[...][None], (BPS, F, C)).reshape(NR, C)
    s = jax.nn.relu(xg * rb)
    y = (jax.nn.relu(lax.dot_general(s, w1v_ref[...], _NT,
                                     preferred_element_type=f32)
                     + b1v_ref[...])
         + lax.dot_general(s, w1s_ref[...], _NT, preferred_element_type=f32)
         + b1s_ref[...])
    z = (jax.nn.relu(lax.dot_general(y, w2v_ref[...], _NT,
                                     preferred_element_type=f32)
                     + b2v_ref[...])
         + lax.dot_general(y, w2s_ref[...], _NT, preferred_element_type=f32)
         + b2s_ref[...])
    scaled = tr_ref[0] / tsr_ref[...]
    pes = jnp.sin(scaled)
    pec = jnp.cos(scaled)
    cidx = lax.broadcasted_iota(jnp.int32, (L, C), 1)
    ridx4 = lax.broadcasted_iota(jnp.int32, (L, C), 0) * D_OB
    planes = [lax.dot_general((cidx == ridx4 + o).astype(f32), z, _NT,
                              preferred_element_type=f32)
              for o in range(D_OB)]
    rr = lax.broadcasted_iota(jnp.int32, (D_MODEL, D_MODEL), 0)
    cc = lax.broadcasted_iota(jnp.int32, (D_MODEL, D_MODEL), 1)
    ecat = (cc == (rr % F) * D_OB + rr // F).astype(f32)
    xbs = []
    for j in range(BPS):
        zb = jnp.concatenate([p[:, j * F:(j + 1) * F] for p in planes], axis=1)
        ub = jnp.dot(zb, ecat, preferred_element_type=f32)
        xbs.append(jnp.concatenate(
            [ub, pes[:, j * NPE:(j + 1) * NPE], pec[:, j * NPE:(j + 1) * NPE]],
            axis=1))
    x = jnp.concatenate(xbs, axis=0)
    scale = 1.0 / float(np.sqrt(HD))
    col = lax.broadcasted_iota(jnp.int32, (1, D), 1)
    masks = [(col // HD == h).astype(f32) for h in range(H)]
    for l in range(N_LAYERS):
        q = lax.dot_general(x, wq_ref[l], _NT,
                            preferred_element_type=f32) + bq_ref[l]
        k = lax.dot_general(x, wk_ref[l], _NT,
                            preferred_element_type=f32) + bk_ref[l]
        v = lax.dot_general(x, wv_ref[l], _NT,
                            preferred_element_type=f32) + bv_ref[l]
        os = []
        for j in range(BPS):
            qj = q[j * L:(j + 1) * L]
            kj = k[j * L:(j + 1) * L]
            vj = v[j * L:(j + 1) * L]
            neg = neg_ref[j]
            kms = jnp.concatenate([kj * mh for mh in masks], axis=0)
            vms = jnp.concatenate([vj * mh for mh in masks], axis=0)
            sc = lax.dot_general(qj, kms, _NT,
                                 preferred_element_type=f32) * scale
            ps = []
            for h in range(H):
                sh = sc[:, h * L:(h + 1) * L] + neg
                m = jnp.max(sh, axis=-1, keepdims=True)
                ex = jnp.exp(sh - m)
                ps.append(ex / jnp.sum(ex, axis=-1, keepdims=True))
            p = jnp.concatenate(ps, axis=1)
            os.append(jnp.dot(p, vms, preferred_element_type=f32))
        o = jnp.concatenate(os, axis=0)
        a = lax.dot_general(o, wo_ref[l], _NT,
                            preferred_element_type=f32) + bo_ref[l]
        x = _ln(x + a, n1w_ref[l], n1b_ref[l])
        ff = lax.dot_general(
            jax.nn.relu(
                lax.dot_general(x, w1_ref[l], _NT, preferred_element_type=f32)
                + b1_ref[l]),
            w2_ref[l], _NT, preferred_element_type=f32) + b2_ref[l]
        x = _ln(x + ff, n2w_ref[l], n2b_ref[l])
    for j in range(BPS):
        out_ref[:, j, :] = x[j * L:(j + 1) * L]


def kernel(X, timestamps, lengths, R_u, op1_vw, op1_vb, op1_sw, op1_sb,
           op2_vw, op2_vb, op2_sw, op2_sb, in_proj_w, in_proj_b,
           out_proj_w, out_proj_b, lin1_w, lin1_b, lin2_w, lin2_b,
           norm1_w, norm1_b, norm2_w, norm2_b):
    f32 = jnp.float32

    xt = X.transpose(0, 2, 1).reshape(B * F, L)
    rp = jnp.broadcast_to(R_u.reshape(F, D_OB)[:, None, :],
                          (F, L, D_OB)).reshape(F, C)
    times_rep = jnp.repeat(
        timestamps.reshape(B // BPS, BPS, L).transpose(0, 2, 1), NPE, axis=2)
    ts_rep = jnp.tile(jnp.asarray(_TIMESCALES).reshape(1, NPE), (1, BPS))

    mask = jnp.arange(L)[None, :] >= lengths
    neg = jnp.where(mask, jnp.float32(-1e30), jnp.float32(0.0))
    neg3 = neg.reshape(B, 1, L)

    wq = in_proj_w[:, 0 * D:1 * D, :]
    wk = in_proj_w[:, 1 * D:2 * D, :]
    wv = in_proj_w[:, 2 * D:3 * D, :]
    bq = in_proj_b[:, 0 * D:1 * D].reshape(N_LAYERS, 1, D)
    bk = in_proj_b[:, 1 * D:2 * D].reshape(N_LAYERS, 1, D)
    bv = in_proj_b[:, 2 * D:3 * D].reshape(N_LAYERS, 1, D)

    full = lambda shape: pl.BlockSpec(shape, lambda b: (0,) * len(shape))
    xout = pl.pallas_call(
        _body,
        grid=(B // BPS,),
        in_specs=[
            pl.BlockSpec((NR, L), lambda b: (b, 0)),
            full((F, C)),
            full((C, C)), full((1, C)), full((C, C)), full((1, C)),
            full((C, C)), full((1, C)), full((C, C)), full((1, C)),
            pl.BlockSpec((1, L, BPS * NPE), lambda b: (b, 0, 0)),
            full((1, BPS * NPE)),
            pl.BlockSpec((BPS, 1, L), lambda b: (b, 0, 0)),
            full((N_LAYERS, D, D)), full((N_LAYERS, D, D)), full((N_LAYERS, D, D)),
            full((N_LAYERS, 1, D)), full((N_LAYERS, 1, D)), full((N_LAYERS, 1, D)),
            full((N_LAYERS, D, D)), full((N_LAYERS, 1, D)),
            full((N_LAYERS, D_FFN, D)), full((N_LAYERS, 1, D_FFN)),
            full((N_LAYERS, D, D_FFN)), full((N_LAYERS, 1, D)),
            full((N_LAYERS, 1, D)), full((N_LAYERS, 1, D)),
            full((N_LAYERS, 1, D)), full((N_LAYERS, 1, D)),
        ],
        out_specs=pl.BlockSpec((L, BPS, D), lambda b: (0, b, 0)),
        out_shape=jax.ShapeDtypeStruct((L, B, D), f32),
    )(xt, rp,
      op1_vw, op1_vb.reshape(1, C), op1_sw, op1_sb.reshape(1, C),
      op2_vw, op2_vb.reshape(1, C), op2_sw, op2_sb.reshape(1, C),
      times_rep, ts_rep, neg3,
      wq, wk, wv, bq, bk, bv,
      out_proj_w, out_proj_b.reshape(N_LAYERS, 1, D),
      lin1_w, lin1_b.reshape(N_LAYERS, 1, D_FFN),
      lin2_w, lin2_b.reshape(N_LAYERS, 1, D),
      norm1_w.reshape(N_LAYERS, 1, D), norm1_b.reshape(N_LAYERS, 1, D),
      norm2_w.reshape(N_LAYERS, 1, D), norm2_b.reshape(N_LAYERS, 1, D))

    return xout, mask


# speedup vs baseline: 1.0508x; 1.0372x over previous
import numpy as np
import jax
import jax.numpy as jnp
from jax import lax
from jax.experimental import pallas as pl

B = 32
L = 128
F = 32
D_OB = 4
D_MODEL = F * D_OB
D_PE = 16
D = D_MODEL + D_PE
H = 12
HD = D // H
D_FFN = 512
N_LAYERS = 2
C = L * D_OB
BPS = 8
NR = BPS * F
NPE = D_PE // 2

_TIMESCALES = np.asarray(float(L) ** np.linspace(0.0, 1.0, NPE),
                         dtype=np.float32)

_NT = (((1,), (1,)), ((), ()))


def _ln(t, w, b):
    mu = jnp.mean(t, axis=-1, keepdims=True)
    var = jnp.mean((t - mu) ** 2, axis=-1, keepdims=True)
    return (t - mu) / jnp.sqrt(var + 1e-5) * w + b


def _body(xt_ref, rp_ref, esel_ref, psel_ref, ecat_ref,
          w1v_ref, b1v_ref, w1s_ref, b1s_ref,
          w2v_ref, b2v_ref, w2s_ref, b2s_ref, tr_ref, tsr_ref, neg_ref,
          wq_ref, wk_ref, wv_ref, bq_ref, bk_ref, bv_ref,
          wo_ref, bo_ref, w1_ref, b1_ref, w2_ref, b2_ref,
          n1w_ref, n1b_ref, n2w_ref, n2b_ref, out_ref):
    f32 = jnp.float32
    xg = jnp.dot(xt_ref[...], esel_ref[...], preferred_element_type=f32)
    rb = jnp.broadcast_to(rp_ref[...][None], (BPS, F, C)).reshape(NR, C)
    s = jax.nn.relu(xg * rb)
    y = (jax.nn.relu(lax.dot_general(s, w1v_ref[...], _NT,
                                     preferred_element_type=f32)
                     + b1v_ref[...])
         + lax.dot_general(s, w1s_ref[...], _NT, preferred_element_type=f32)
         + b1s_ref[...])
    z = (jax.nn.relu(lax.dot_general(y, w2v_ref[...], _NT,
                                     preferred_element_type=f32)
                     + b2v_ref[...])
         + lax.dot_general(y, w2s_ref[...], _NT, preferred_element_type=f32)
         + b2s_ref[...])
    scaled = tr_ref[0] / tsr_ref[...]
    pes = jnp.sin(scaled)
    pec = jnp.cos(scaled)
    planes = [lax.dot_general(psel_ref[o], z, _NT, preferred_element_type=f32)
              for o in range(D_OB)]
    xbs = []
    for j in range(BPS):
        zb = jnp.concatenate([p[:, j * F:(j + 1) * F] for p in planes], axis=1)
        ub = jnp.dot(zb, ecat_ref[...], preferred_element_type=f32)
        xbs.append(jnp.concatenate(
            [ub, pes[:, j * NPE:(j + 1) * NPE], pec[:, j * NPE:(j + 1) * NPE]],
            axis=1))
    x = jnp.concatenate(xbs, axis=0)
    scale = 1.0 / float(np.sqrt(HD))
    col = lax.broadcasted_iota(jnp.int32, (1, D), 1)
    masks = [(col // HD == h).astype(f32) for h in range(H)]
    for l in range(N_LAYERS):
        q = lax.dot_general(x, wq_ref[l], _NT,
                            preferred_element_type=f32) + bq_ref[l]
        k = lax.dot_general(x, wk_ref[l], _NT,
                            preferred_element_type=f32) + bk_ref[l]
        v = lax.dot_general(x, wv_ref[l], _NT,
                            preferred_element_type=f32) + bv_ref[l]
        os = []
        for j in range(BPS):
            qj = q[j * L:(j + 1) * L]
            kj = k[j * L:(j + 1) * L]
            vj = v[j * L:(j + 1) * L]
            neg = neg_ref[j]
            qms = jnp.concatenate([qj * mh for mh in masks], axis=0)
            sc = lax.dot_general(qms, kj, _NT,
                                 preferred_element_type=f32) * scale
            sc = sc + neg
            m = jnp.max(sc, axis=-1, keepdims=True)
            ex = jnp.exp(sc - m)
            p = ex / jnp.sum(ex, axis=-1, keepdims=True)
            r = jnp.dot(p, vj, preferred_element_type=f32)
            oj = r[0:L] * masks[0]
            for h in range(1, H):
                oj = oj + r[h * L:(h + 1) * L] * masks[h]
            os.append(oj)
        o = jnp.concatenate(os, axis=0)
        a = lax.dot_general(o, wo_ref[l], _NT,
                            preferred_element_type=f32) + bo_ref[l]
        x = _ln(x + a, n1w_ref[l], n1b_ref[l])
        ff = lax.dot_general(
            jax.nn.relu(
                lax.dot_general(x, w1_ref[l], _NT, preferred_element_type=f32)
                + b1_ref[l]),
            w2_ref[l], _NT, preferred_element_type=f32) + b2_ref[l]
        x = _ln(x + ff, n2w_ref[l], n2b_ref[l])
    for j in range(BPS):
        out_ref[:, j, :] = x[j * L:(j + 1) * L]


def kernel(X, timestamps, lengths, R_u, op1_vw, op1_vb, op1_sw, op1_sb,
           op2_vw, op2_vb, op2_sw, op2_sb, in_proj_w, in_proj_b,
           out_proj_w, out_proj_b, lin1_w, lin1_b, lin2_w, lin2_b,
           norm1_w, norm1_b, norm2_w, norm2_b):
    f32 = jnp.float32

    xt = X.transpose(0, 2, 1).reshape(B * F, L)
    rp = jnp.broadcast_to(R_u.reshape(F, D_OB)[:, None, :],
                          (F, L, D_OB)).reshape(F, C)
    times_rep = jnp.repeat(
        timestamps.reshape(B // BPS, BPS, L).transpose(0, 2, 1), NPE, axis=2)
    ts_rep = jnp.tile(jnp.asarray(_TIMESCALES).reshape(1, NPE), (1, BPS))

    li = jnp.arange(L)[:, None]
    ci = jnp.arange(C)[None, :]
    esel = (ci // D_OB == li).astype(f32)
    psel = jnp.stack([(ci == li * D_OB + o).astype(f32)
                      for o in range(D_OB)], axis=0)
    ri = jnp.arange(D_MODEL)[:, None]
    ecat = (jnp.arange(D_MODEL)[None, :]
            == (ri % F) * D_OB + ri // F).astype(f32)

    mask = jnp.arange(L)[None, :] >= lengths
    neg = jnp.where(mask, jnp.float32(-1e30), jnp.float32(0.0))
    neg3 = neg.reshape(B, 1, L)

    wq = in_proj_w[:, 0 * D:1 * D, :]
    wk = in_proj_w[:, 1 * D:2 * D, :]
    wv = in_proj_w[:, 2 * D:3 * D, :]
    bq = in_proj_b[:, 0 * D:1 * D].reshape(N_LAYERS, 1, D)
    bk = in_proj_b[:, 1 * D:2 * D].reshape(N_LAYERS, 1, D)
    bv = in_proj_b[:, 2 * D:3 * D].reshape(N_LAYERS, 1, D)

    full = lambda shape: pl.BlockSpec(shape, lambda b: (0,) * len(shape))
    xout = pl.pallas_call(
        _body,
        grid=(B // BPS,),
        in_specs=[
            pl.BlockSpec((NR, L), lambda b: (b, 0)),
            full((F, C)),
            full((L, C)), full((D_OB, L, C)), full((D_MODEL, D_MODEL)),
            full((C, C)), full((1, C)), full((C, C)), full((1, C)),
            full((C, C)), full((1, C)), full((C, C)), full((1, C)),
            pl.BlockSpec((1, L, BPS * NPE), lambda b: (b, 0, 0)),
            full((1, BPS * NPE)),
            pl.BlockSpec((BPS, 1, L), lambda b: (b, 0, 0)),
            full((N_LAYERS, D, D)), full((N_LAYERS, D, D)), full((N_LAYERS, D, D)),
            full((N_LAYERS, 1, D)), full((N_LAYERS, 1, D)), full((N_LAYERS, 1, D)),
            full((N_LAYERS, D, D)), full((N_LAYERS, 1, D)),
            full((N_LAYERS, D_FFN, D)), full((N_LAYERS, 1, D_FFN)),
            full((N_LAYERS, D, D_FFN)), full((N_LAYERS, 1, D)),
            full((N_LAYERS, 1, D)), full((N_LAYERS, 1, D)),
            full((N_LAYERS, 1, D)), full((N_LAYERS, 1, D)),
        ],
        out_specs=pl.BlockSpec((L, BPS, D), lambda b: (0, b, 0)),
        out_shape=jax.ShapeDtypeStruct((L, B, D), f32),
    )(xt, rp, esel, psel, ecat,
      op1_vw, op1_vb.reshape(1, C), op1_sw, op1_sb.reshape(1, C),
      op2_vw, op2_vb.reshape(1, C), op2_sw, op2_sb.reshape(1, C),
      times_rep, ts_rep, neg3,
      wq, wk, wv, bq, bk, bv,
      out_proj_w, out_proj_b.reshape(N_LAYERS, 1, D),
      lin1_w, lin1_b.reshape(N_LAYERS, 1, D_FFN),
      lin2_w, lin2_b.reshape(N_LAYERS, 1, D),
      norm1_w.reshape(N_LAYERS, 1, D), norm1_b.reshape(N_LAYERS, 1, D),
      norm2_w.reshape(N_LAYERS, 1, D), norm2_b.reshape(N_LAYERS, 1, D))

    return xout, mask


# speedup vs baseline: 1.0521x; 1.0013x over previous
import numpy as np
import jax
import jax.numpy as jnp
from jax import lax
from jax.experimental import pallas as pl

B = 32
L = 128
F = 32
D_OB = 4
D_MODEL = F * D_OB
D_PE = 16
D = D_MODEL + D_PE
H = 12
HD = D // H
D_FFN = 512
N_LAYERS = 2
C = L * D_OB
BPS = 8
NR = BPS * F
NPE = D_PE // 2

_TIMESCALES = np.asarray(float(L) ** np.linspace(0.0, 1.0, NPE),
                         dtype=np.float32)

_NT = (((1,), (1,)), ((), ()))


def _ln(t, w, b):
    mu = jnp.mean(t, axis=-1, keepdims=True)
    var = jnp.mean((t - mu) ** 2, axis=-1, keepdims=True)
    return (t - mu) / jnp.sqrt(var + 1e-5) * w + b


def _body(xt_ref, rp_ref, esel_ref, psel_ref, ecat_ref,
          w1v_ref, b1v_ref, w1s_ref, b1s_ref,
          w2v_ref, b2v_ref, w2s_ref, b2s_ref, tr_ref, tsr_ref, neg_ref,
          wq_ref, bq_ref, bk_ref, bv_ref,
          wo_ref, bo_ref, w1_ref, b1_ref, w2_ref, b2_ref,
          n1w_ref, n1b_ref, n2w_ref, n2b_ref, out_ref):
    f32 = jnp.float32
    xg = jnp.dot(xt_ref[...], esel_ref[...], preferred_element_type=f32)
    rb = jnp.broadcast_to(rp_ref[...][None], (BPS, F, C)).reshape(NR, C)
    s = jax.nn.relu(xg * rb)
    y = (jax.nn.relu(lax.dot_general(s, w1v_ref[...], _NT,
                                     preferred_element_type=f32)
                     + b1v_ref[...])
         + lax.dot_general(s, w1s_ref[...], _NT, preferred_element_type=f32)
         + b1s_ref[...])
    z = (jax.nn.relu(lax.dot_general(y, w2v_ref[...], _NT,
                                     preferred_element_type=f32)
                     + b2v_ref[...])
         + lax.dot_general(y, w2s_ref[...], _NT, preferred_element_type=f32)
         + b2s_ref[...])
    scaled = tr_ref[0] / tsr_ref[...]
    pes = jnp.sin(scaled)
    pec = jnp.cos(scaled)
    pall = lax.dot_general(psel_ref[...], z, _NT,
                           preferred_element_type=f32)
    planes = [pall[o * L:(o + 1) * L] for o in range(D_OB)]
    xbs = []
    for j in range(BPS):
        zb = jnp.concatenate([p[:, j * F:(j + 1) * F] for p in planes], axis=1)
        ub = jnp.dot(zb, ecat_ref[...], preferred_element_type=f32)
        xbs.append(jnp.concatenate(
            [ub, pes[:, j * NPE:(j + 1) * NPE], pec[:, j * NPE:(j + 1) * NPE]],
            axis=1))
    x = jnp.concatenate(xbs, axis=0)
    scale = 1.0 / float(np.sqrt(HD))
    col = lax.broadcasted_iota(jnp.int32, (1, D), 1)
    masks = [(col // HD == h).astype(f32) for h in range(H)]
    for l in range(N_LAYERS):
        w3 = wq_ref[l]
        q = lax.dot_general(x, w3[0 * D:1 * D], _NT,
                            preferred_element_type=f32) + bq_ref[l]
        k = lax.dot_general(x, w3[1 * D:2 * D], _NT,
                            preferred_element_type=f32) + bk_ref[l]
        v = lax.dot_general(x, w3[2 * D:3 * D], _NT,
                            preferred_element_type=f32) + bv_ref[l]
        os = []
        for j in range(BPS):
            qj = q[j * L:(j + 1) * L]
            kj = k[j * L:(j + 1) * L]
            vj = v[j * L:(j + 1) * L]
            neg = neg_ref[j]
            qms = jnp.concatenate([qj * mh for mh in masks], axis=0)
            sc = lax.dot_general(qms, kj, _NT,
                                 preferred_element_type=f32) * scale
            sc = sc + neg
            m = jnp.max(sc, axis=-1, keepdims=True)
            ex = jnp.exp(sc - m)
            p = ex / jnp.sum(ex, axis=-1, keepdims=True)
            r = jnp.dot(p, vj, preferred_element_type=f32)
            oj = r[0:L] * masks[0]
            for h in range(1, H):
                oj = oj + r[h * L:(h + 1) * L] * masks[h]
            os.append(oj)
        o = jnp.concatenate(os, axis=0)
        a = lax.dot_general(o, wo_ref[l], _NT,
                            preferred_element_type=f32) + bo_ref[l]
        x = _ln(x + a, n1w_ref[l], n1b_ref[l])
        ff = lax.dot_general(
            jax.nn.relu(
                lax.dot_general(x, w1_ref[l], _NT, preferred_element_type=f32)
                + b1_ref[l]),
            w2_ref[l], _NT, preferred_element_type=f32) + b2_ref[l]
        x = _ln(x + ff, n2w_ref[l], n2b_ref[l])
    for j in range(BPS):
        out_ref[:, j, :] = x[j * L:(j + 1) * L]


def kernel(X, timestamps, lengths, R_u, op1_vw, op1_vb, op1_sw, op1_sb,
           op2_vw, op2_vb, op2_sw, op2_sb, in_proj_w, in_proj_b,
           out_proj_w, out_proj_b, lin1_w, lin1_b, lin2_w, lin2_b,
           norm1_w, norm1_b, norm2_w, norm2_b):
    f32 = jnp.float32

    xt = X.transpose(0, 2, 1).reshape(B * F, L)
    rp = jnp.broadcast_to(R_u.reshape(F, D_OB)[:, None, :],
                          (F, L, D_OB)).reshape(F, C)
    times_rep = jnp.repeat(
        timestamps.reshape(B // BPS, BPS, L).transpose(0, 2, 1), NPE, axis=2)
    ts_rep = jnp.tile(jnp.asarray(_TIMESCALES).reshape(1, NPE), (1, BPS))

    li = jnp.arange(L)[:, None]
    ci = jnp.arange(C)[None, :]
    esel = (ci // D_OB == li).astype(f32)
    psel = jnp.concatenate([(ci == li * D_OB + o).astype(f32)
                            for o in range(D_OB)], axis=0)
    ri = jnp.arange(D_MODEL)[:, None]
    ecat = (jnp.arange(D_MODEL)[None, :]
            == (ri % F) * D_OB + ri // F).astype(f32)

    mask = jnp.arange(L)[None, :] >= lengths
    neg = jnp.where(mask, jnp.float32(-1e30), jnp.float32(0.0))
    neg3 = neg.reshape(B, 1, L)

    bq = in_proj_b[:, 0 * D:1 * D].reshape(N_LAYERS, 1, D)
    bk = in_proj_b[:, 1 * D:2 * D].reshape(N_LAYERS, 1, D)
    bv = in_proj_b[:, 2 * D:3 * D].reshape(N_LAYERS, 1, D)

    full = lambda shape: pl.BlockSpec(shape, lambda b: (0,) * len(shape))
    xout = pl.pallas_call(
        _body,
        grid=(B // BPS,),
        in_specs=[
            pl.BlockSpec((NR, L), lambda b: (b, 0)),
            full((F, C)),
            full((L, C)), full((D_OB * L, C)), full((D_MODEL, D_MODEL)),
            full((C, C)), full((1, C)), full((C, C)), full((1, C)),
            full((C, C)), full((1, C)), full((C, C)), full((1, C)),
            pl.BlockSpec((1, L, BPS * NPE), lambda b: (b, 0, 0)),
            full((1, BPS * NPE)),
            pl.BlockSpec((BPS, 1, L), lambda b: (b, 0, 0)),
            full((N_LAYERS, 3 * D, D)),
            full((N_LAYERS, 1, D)), full((N_LAYERS, 1, D)), full((N_LAYERS, 1, D)),
            full((N_LAYERS, D, D)), full((N_LAYERS, 1, D)),
            full((N_LAYERS, D_FFN, D)), full((N_LAYERS, 1, D_FFN)),
            full((N_LAYERS, D, D_FFN)), full((N_LAYERS, 1, D)),
            full((N_LAYERS, 1, D)), full((N_LAYERS, 1, D)),
            full((N_LAYERS, 1, D)), full((N_LAYERS, 1, D)),
        ],
        out_specs=pl.BlockSpec((L, BPS, D), lambda b: (0, b, 0)),
        out_shape=jax.ShapeDtypeStruct((L, B, D), f32),
    )(xt, rp, esel, psel, ecat,
      op1_vw, op1_vb.reshape(1, C), op1_sw, op1_sb.reshape(1, C),
      op2_vw, op2_vb.reshape(1, C), op2_sw, op2_sb.reshape(1, C),
      times_rep, ts_rep, neg3,
      in_proj_w, bq, bk, bv,
      out_proj_w, out_proj_b.reshape(N_LAYERS, 1, D),
      lin1_w, lin1_b.reshape(N_LAYERS, 1, D_FFN),
      lin2_w, lin2_b.reshape(N_LAYERS, 1, D),
      norm1_w.reshape(N_LAYERS, 1, D), norm1_b.reshape(N_LAYERS, 1, D),
      norm2_w.reshape(N_LAYERS, 1, D), norm2_b.reshape(N_LAYERS, 1, D))

    return xout, mask
